# baseline (speedup 1.0000x reference)
import functools

import jax
import jax.numpy as jnp
from jax import lax
from jax.experimental import pallas as pl
from jax.experimental.pallas import tpu as pltpu
from jax.experimental.pallas import tpu_sc as plsc

_M = 1024
_K = 16
_MC = 256
_NB = 32
_BIG = 3.0e38


def _tc_body(coords_ref, nodes_ref, feats_ref, s_ref, t_ref, ft_ref):
    x = coords_ref[0]
    nd = nodes_ref[0]
    N = x.shape[1]
    @pl.when(pl.program_id(1) == 0)
    def _():
        ft_ref[0] = jnp.transpose(feats_ref[0], (1, 0))

    s = None
    for c in range(3):
        d = x[c : c + 1, :] - nd[:, c : c + 1]
        s = d * d if s is None else s + d * d
    s_ref[0] = s

    w = N // _NB
    bm = jnp.concatenate(
        [jnp.min(s[:, g * w : (g + 1) * w], axis=1, keepdims=True)
         for g in range(_NB)], axis=1)
    for _ in range(_K - 1):
        m = jnp.min(bm, axis=1, keepdims=True)
        bm = jnp.where(bm <= m, _BIG, bm)
    t = jnp.min(bm, axis=1, keepdims=True)
    t_ref[0] = jnp.broadcast_to(t, (t.shape[0], 16))


def _sc_kernel(total_rows, n, c_feat):
    info = plsc.get_sparse_core_info()
    nw = info.num_cores * info.num_subcores
    rpw = total_rows // nw
    nchunks = n // 16

    mesh = plsc.VectorSubcoreMesh(core_axis_name="c", subcore_axis_name="s")

    @functools.partial(
        pl.kernel,
        mesh=mesh,
        out_type=jax.ShapeDtypeStruct(
            (total_rows // _M, 2 * c_feat, _M), jnp.float32),
        compiler_params=pltpu.CompilerParams(
            needs_layout_passes=False, use_tc_tiling_on_sc=False),
        scratch_types=[
            pltpu.VMEM((rpw, 16), jnp.float32),
            pltpu.VMEM((2, n), jnp.float32),
            pltpu.VMEM((n,), jnp.float32),
            pltpu.VMEM((n,), jnp.int32),
            pltpu.VMEM((nchunks,), jnp.int32),
            pltpu.VMEM((rpw * 16,), jnp.int32),
            pltpu.VMEM((2, 512, 64), jnp.float32),
            pltpu.VMEM((c_feat, rpw), jnp.float32),
            pltpu.VMEM((c_feat, rpw), jnp.float32),
            pltpu.SemaphoreType.DMA,
            pltpu.SemaphoreType.DMA,
        ],
    )
    def body(s_hbm, t_hbm, ft_hbm, feats_hbm, out_hbm,
             t_v, srow, cand, cidx, hitl, idxall, grows, out_v, fraw,
             sem, sem2):
        wid = lax.axis_index("s") * info.num_cores + lax.axis_index("c")
        base = wid * rpw
        bb = base // _M
        col0 = base - bb * _M
        pltpu.sync_copy(t_hbm.at[pl.ds(base, rpw)], t_v)
        pltpu.sync_copy(
            feats_hbm.at[bb, :, pl.ds(col0, rpw)], fraw)
        pltpu.sync_copy(
            fraw, out_hbm.at[bb, pl.ds(0, c_feat), pl.ds(col0, rpw)])
        lanes = lax.iota(jnp.int32, 16)
        ngroups = nchunks // 16
        pltpu.async_copy(s_hbm.at[base], srow.at[0], sem)

        def per_row(r, _):
            buf = lax.rem(r, 2)
            pltpu.make_async_copy(
                s_hbm.at[base + r], srow.at[buf], sem).wait()

            @pl.when(r + 1 < rpw)
            def _():
                pltpu.async_copy(
                    s_hbm.at[base + r + 1], srow.at[1 - buf], sem)

            tb = t_v[r]

            def count_group(g, hcnt):
                acc = jnp.zeros((16,), jnp.int32)
                for j in range(16):
                    v = srow[buf, pl.ds(g * 256 + j * 16, 16)]
                    msk = v <= tb
                    pc = plsc.all_reduce_population_count(msk)
                    acc = acc + jnp.where(lanes == j, pc, 0)
                gm = acc > 0
                ranks = plsc.cumsum(jnp.where(gm, 1, 0))
                dst = hcnt + ranks - 1
                plsc.store_scatter(hitl, [dst], g * 16 + lanes, mask=gm)
                return hcnt + plsc.all_reduce_population_count(gm)

            hcnt = lax.fori_loop(
                0, ngroups, count_group, jnp.zeros((16,), jnp.int32))
            nh = jnp.max(hcnt)

            def hit_chunk(j, cnt):
                cid = plsc.load_gather(hitl, [jnp.full((16,), j, jnp.int32)])
                vidx = cid * 16 + lanes
                v = plsc.load_gather(srow.at[buf], [vidx])
                msk = v <= tb
                ranks = plsc.cumsum(jnp.where(msk, 1, 0))
                dst = cnt + ranks - 1
                plsc.store_scatter(cand, [dst], v, mask=msk)
                plsc.store_scatter(cidx, [dst], vidx, mask=msk)
                return cnt + plsc.all_reduce_population_count(msk)

            cnt = lax.fori_loop(
                0, nh, hit_chunk, jnp.zeros((16,), jnp.int32))
            cnt_s = jnp.max(cnt)
            nmerge = (cnt_s + 15) // 16

            def merge(j, carry):
                bv, bi = carry
                v = cand[pl.ds(j * 16, 16)]
                ci = cidx[pl.ds(j * 16, 16)]
                valid = (lanes + j * 16) < cnt
                v = jnp.where(valid, v, _BIG)
                sv, si = plsc.sort_key_val(v, ci)
                rv = lax.rev(sv, (0,))
                ri = lax.rev(si, (0,))
                take = rv < bv
                nv = jnp.where(take, rv, bv)
                ni = jnp.where(take, ri, bi)
                return tuple(plsc.sort_key_val(nv, ni))

            best0 = (jnp.full((16,), _BIG, jnp.float32),
                     jnp.zeros((16,), jnp.int32))
            _, bi = lax.fori_loop(0, nmerge, merge, best0)

            b = (base + r) // _M
            idxall[pl.ds(r * 16, 16)] = bi + b * n
            return 0

        lax.fori_loop(0, rpw, per_row, 0)

        npb = 32
        nbat = rpw // npb
        pltpu.async_copy(
            ft_hbm.at[idxall.at[pl.ds(0, npb * 16)]], grows.at[0], sem2)

        def gbatch(g, _):
            gb = lax.rem(g, 2)
            pltpu.make_async_copy(
                ft_hbm.at[idxall.at[pl.ds(g * npb * 16, npb * 16)]],
                grows.at[gb], sem2).wait()

            @pl.when(g + 1 < nbat)
            def _():
                pltpu.async_copy(
                    ft_hbm.at[idxall.at[pl.ds((g + 1) * npb * 16, npb * 16)]],
                    grows.at[1 - gb], sem2)

            def node(i, _):
                nd_ = g * npb + i
                for cb in range(64 // 16):
                    acc = grows[gb, i * 16, pl.ds(cb * 16, 16)]
                    for q in range(1, 16):
                        acc = jnp.maximum(
                            acc, grows[gb, i * 16 + q, pl.ds(cb * 16, 16)])
                    plsc.store_scatter(
                        out_v, [cb * 16 + lanes,
                                jnp.full((16,), nd_, jnp.int32)], acc)
                return 0

            lax.fori_loop(0, npb, node, 0)
            return 0

        lax.fori_loop(0, nbat, gbatch, 0)
        pltpu.sync_copy(
            out_v, out_hbm.at[bb, pl.ds(c_feat, c_feat), pl.ds(col0, rpw)])

    return body


@jax.jit
def kernel(input_coords, input_feats):
    B, _, N = input_coords.shape
    C = input_feats.shape[1]
    coords_p = jnp.pad(input_coords, ((0, 0), (0, 5), (0, 0)))
    nodes_t = jnp.transpose(coords_p[:, :, :_M], (0, 2, 1))

    grid = (B, _M // _MC)
    s_all, t_all, ft_all = pl.pallas_call(
        _tc_body,
        grid=grid,
        in_specs=[
            pl.BlockSpec((1, 8, N), lambda b, m: (b, 0, 0)),
            pl.BlockSpec((1, _MC, 8), lambda b, m: (b, m, 0)),
            pl.BlockSpec((1, C, N), lambda b, m: (b, 0, 0)),
        ],
        out_specs=[
            pl.BlockSpec((1, _MC, N), lambda b, m: (b, m, 0)),
            pl.BlockSpec((1, _MC, 16), lambda b, m: (b, m, 0)),
            pl.BlockSpec((1, N, C), lambda b, m: (b, 0, 0)),
        ],
        out_shape=[
            jax.ShapeDtypeStruct((B, _M, N), jnp.float32),
            jax.ShapeDtypeStruct((B, _M, 16), jnp.float32),
            jax.ShapeDtypeStruct((B, N, C), jnp.float32),
        ],
    )(coords_p, nodes_t, input_feats)

    total = B * _M
    s_flat = s_all.reshape(total, N)
    t_flat = t_all.reshape(total, 16)
    ft_flat = ft_all.reshape(B * N, C)

    pool_feats = _sc_kernel(total, N, C)(
        s_flat, t_flat, ft_flat, input_feats)

    pool_coords = input_coords[:, :, :_M]
    return (pool_coords, pool_coords, pool_feats)


# speedup vs baseline: 1.0313x; 1.0313x over previous
import functools

import jax
import jax.numpy as jnp
from jax import lax
from jax.experimental import pallas as pl
from jax.experimental.pallas import tpu as pltpu
from jax.experimental.pallas import tpu_sc as plsc

_M = 1024
_K = 16
_MC = 512
_NB = 32
_BIG = 3.0e38


def _tc_body(coords_ref, nodes_ref, feats_ref, s_ref, t_ref, ft_ref):
    x = coords_ref[0]
    nd = nodes_ref[0]
    N = x.shape[1]
    @pl.when(pl.program_id(1) == 0)
    def _():
        ft_ref[0] = jnp.transpose(feats_ref[0], (1, 0))

    s = None
    for c in range(3):
        d = x[c : c + 1, :] - nd[:, c : c + 1]
        s = d * d if s is None else s + d * d
    s_ref[0] = s

    w = N // _NB
    bm = jnp.concatenate(
        [jnp.min(s[:, g * w : (g + 1) * w], axis=1, keepdims=True)
         for g in range(_NB)], axis=1)
    for _ in range(_K - 1):
        m = jnp.min(bm, axis=1, keepdims=True)
        bm = jnp.where(bm <= m, _BIG, bm)
    t = jnp.min(bm, axis=1, keepdims=True)
    t_ref[0] = jnp.broadcast_to(t, (t.shape[0], 16))


def _sc_kernel(total_rows, n, c_feat):
    info = plsc.get_sparse_core_info()
    nw = info.num_cores * info.num_subcores
    rpw = total_rows // nw
    nchunks = n // 16

    mesh = plsc.VectorSubcoreMesh(core_axis_name="c", subcore_axis_name="s")

    @functools.partial(
        pl.kernel,
        mesh=mesh,
        out_type=jax.ShapeDtypeStruct(
            (total_rows // _M, 2 * c_feat, _M), jnp.float32),
        compiler_params=pltpu.CompilerParams(
            needs_layout_passes=False, use_tc_tiling_on_sc=False),
        scratch_types=[
            pltpu.VMEM((rpw, 16), jnp.float32),
            pltpu.VMEM((2, n), jnp.float32),
            pltpu.VMEM((n,), jnp.float32),
            pltpu.VMEM((n,), jnp.int32),
            pltpu.VMEM((nchunks,), jnp.int32),
            pltpu.VMEM((rpw * 16,), jnp.int32),
            pltpu.VMEM((2, 512, 64), jnp.float32),
            pltpu.VMEM((c_feat, rpw), jnp.float32),
            pltpu.VMEM((c_feat, rpw), jnp.float32),
            pltpu.SemaphoreType.DMA,
            pltpu.SemaphoreType.DMA,
        ],
    )
    def body(s_hbm, t_hbm, ft_hbm, feats_hbm, out_hbm,
             t_v, srow, cand, cidx, hitl, idxall, grows, out_v, fraw,
             sem, sem2):
        wid = lax.axis_index("s") * info.num_cores + lax.axis_index("c")
        base = wid * rpw
        bb = base // _M
        col0 = base - bb * _M
        pltpu.sync_copy(t_hbm.at[pl.ds(base, rpw)], t_v)
        pltpu.sync_copy(
            feats_hbm.at[bb, :, pl.ds(col0, rpw)], fraw)
        pltpu.sync_copy(
            fraw, out_hbm.at[bb, pl.ds(0, c_feat), pl.ds(col0, rpw)])
        lanes = lax.iota(jnp.int32, 16)
        ngroups = nchunks // 16
        pltpu.async_copy(s_hbm.at[base], srow.at[0], sem)

        def per_row(r, _):
            buf = lax.rem(r, 2)
            pltpu.make_async_copy(
                s_hbm.at[base + r], srow.at[buf], sem).wait()

            @pl.when(r + 1 < rpw)
            def _():
                pltpu.async_copy(
                    s_hbm.at[base + r + 1], srow.at[1 - buf], sem)

            tb = t_v[r]

            def count_group(g, hcnt):
                acc = jnp.zeros((16,), jnp.int32)
                for j in range(16):
                    v = srow[buf, pl.ds(g * 256 + j * 16, 16)]
                    msk = v <= tb
                    pc = plsc.all_reduce_population_count(msk)
                    acc = acc + jnp.where(lanes == j, pc, 0)
                gm = acc > 0
                ranks = plsc.cumsum(jnp.where(gm, 1, 0))
                dst = hcnt + ranks - 1
                plsc.store_scatter(hitl, [dst], g * 16 + lanes, mask=gm)
                return hcnt + plsc.all_reduce_population_count(gm)

            hcnt = lax.fori_loop(
                0, ngroups, count_group, jnp.zeros((16,), jnp.int32))
            nh = jnp.max(hcnt)

            def hit_chunk(j, cnt):
                cid = plsc.load_gather(hitl, [jnp.full((16,), j, jnp.int32)])
                vidx = cid * 16 + lanes
                v = plsc.load_gather(srow.at[buf], [vidx])
                msk = v <= tb
                ranks = plsc.cumsum(jnp.where(msk, 1, 0))
                dst = cnt + ranks - 1
                plsc.store_scatter(cand, [dst], v, mask=msk)
                plsc.store_scatter(cidx, [dst], vidx, mask=msk)
                return cnt + plsc.all_reduce_population_count(msk)

            cnt = lax.fori_loop(
                0, nh, hit_chunk, jnp.zeros((16,), jnp.int32))
            cnt_s = jnp.max(cnt)
            nmerge = (cnt_s + 15) // 16

            def merge(j, carry):
                bv, bi = carry
                v = cand[pl.ds(j * 16, 16)]
                ci = cidx[pl.ds(j * 16, 16)]
                valid = (lanes + j * 16) < cnt
                v = jnp.where(valid, v, _BIG)
                sv, si = plsc.sort_key_val(v, ci)
                rv = lax.rev(sv, (0,))
                ri = lax.rev(si, (0,))
                take = rv < bv
                nv = jnp.where(take, rv, bv)
                ni = jnp.where(take, ri, bi)
                return tuple(plsc.sort_key_val(nv, ni))

            best0 = (jnp.full((16,), _BIG, jnp.float32),
                     jnp.zeros((16,), jnp.int32))
            _, bi = lax.fori_loop(0, nmerge, merge, best0)

            b = (base + r) // _M
            idxall[pl.ds(r * 16, 16)] = bi + b * n
            return 0

        lax.fori_loop(0, rpw, per_row, 0)

        npb = 32
        nbat = rpw // npb
        pltpu.async_copy(
            ft_hbm.at[idxall.at[pl.ds(0, npb * 16)]], grows.at[0], sem2)

        def gbatch(g, _):
            gb = lax.rem(g, 2)
            pltpu.make_async_copy(
                ft_hbm.at[idxall.at[pl.ds(g * npb * 16, npb * 16)]],
                grows.at[gb], sem2).wait()

            @pl.when(g + 1 < nbat)
            def _():
                pltpu.async_copy(
                    ft_hbm.at[idxall.at[pl.ds((g + 1) * npb * 16, npb * 16)]],
                    grows.at[1 - gb], sem2)

            def node(i, _):
                nd_ = g * npb + i
                for cb in range(64 // 16):
                    acc = grows[gb, i * 16, pl.ds(cb * 16, 16)]
                    for q in range(1, 16):
                        acc = jnp.maximum(
                            acc, grows[gb, i * 16 + q, pl.ds(cb * 16, 16)])
                    plsc.store_scatter(
                        out_v, [cb * 16 + lanes,
                                jnp.full((16,), nd_, jnp.int32)], acc)
                return 0

            lax.fori_loop(0, npb, node, 0)
            return 0

        lax.fori_loop(0, nbat, gbatch, 0)
        pltpu.sync_copy(
            out_v, out_hbm.at[bb, pl.ds(c_feat, c_feat), pl.ds(col0, rpw)])

    return body


@jax.jit
def kernel(input_coords, input_feats):
    B, _, N = input_coords.shape
    C = input_feats.shape[1]
    coords_p = jnp.pad(input_coords, ((0, 0), (0, 5), (0, 0)))
    nodes_t = jnp.transpose(coords_p[:, :, :_M], (0, 2, 1))

    grid = (B, _M // _MC)
    s_all, t_all, ft_all = pl.pallas_call(
        _tc_body,
        grid=grid,
        in_specs=[
            pl.BlockSpec((1, 8, N), lambda b, m: (b, 0, 0)),
            pl.BlockSpec((1, _MC, 8), lambda b, m: (b, m, 0)),
            pl.BlockSpec((1, C, N), lambda b, m: (b, 0, 0)),
        ],
        out_specs=[
            pl.BlockSpec((1, _MC, N), lambda b, m: (b, m, 0)),
            pl.BlockSpec((1, _MC, 16), lambda b, m: (b, m, 0)),
            pl.BlockSpec((1, N, C), lambda b, m: (b, 0, 0)),
        ],
        out_shape=[
            jax.ShapeDtypeStruct((B, _M, N), jnp.float32),
            jax.ShapeDtypeStruct((B, _M, 16), jnp.float32),
            jax.ShapeDtypeStruct((B, N, C), jnp.float32),
        ],
    )(coords_p, nodes_t, input_feats)

    total = B * _M
    s_flat = s_all.reshape(total, N)
    t_flat = t_all.reshape(total, 16)
    ft_flat = ft_all.reshape(B * N, C)

    pool_feats = _sc_kernel(total, N, C)(
        s_flat, t_flat, ft_flat, input_feats)

    pool_coords = input_coords[:, :, :_M]
    return (pool_coords, pool_coords, pool_feats)
